# baseline (speedup 1.0000x reference)
import functools
import jax
import jax.numpy as jnp
from jax.experimental import pallas as pl

_TYPES = ['tag', 'module', 'question', 'answer', 'comment']
_NN = {'tag': 10000, 'module': 5000, 'question': 50000, 'answer': 50000,
       'comment': 50000}
_RELS = [('tag', 'question'), ('tag', 'answer'), ('tag', 'comment'),
         ('module', 'question'), ('module', 'answer'), ('question', 'tag'),
         ('answer', 'tag'), ('comment', 'tag'), ('question', 'module'),
         ('answer', 'module')]
_E = 50000
_DF = 128
_H = 64
_B = 1024
_OUT = 2
_L = 2


def _rup(x, m):
    return (x + m - 1) // m * m


def _pad_rows(x, m):
    return jnp.pad(x, ((0, m - x.shape[0]), (0, 0)))


def _leaky(y):
    return jnp.where(y >= 0, y, 0.01 * y)


def _mm_kernel(x_ref, w_ref, b_ref, o_ref, *, leaky):
    y = jnp.dot(x_ref[...], w_ref[...], preferred_element_type=jnp.float32)
    y = y + b_ref[...]
    if leaky:
        y = _leaky(y)
    o_ref[...] = y


def _mm(x, w, b=None, leaky=False, bm=2048):
    m, k = x.shape
    n = w.shape[1]
    mp = _rup(m, bm)
    xp = _pad_rows(x, mp)
    if b is None:
        b = jnp.zeros((n,), jnp.float32)
    b2 = b.reshape(1, n)
    out = pl.pallas_call(
        functools.partial(_mm_kernel, leaky=leaky),
        grid=(mp // bm,),
        in_specs=[pl.BlockSpec((bm, k), lambda i: (i, 0)),
                  pl.BlockSpec((k, n), lambda i: (0, 0)),
                  pl.BlockSpec((1, n), lambda i: (0, 0))],
        out_specs=pl.BlockSpec((bm, n), lambda i: (i, 0)),
        out_shape=jax.ShapeDtypeStruct((mp, n), jnp.float32),
    )(xp, w, b2)
    return out[:m]


def _gather_kernel(idx_ref, t_ref, o_ref, *, be, bn):
    j = pl.program_id(1)

    @pl.when(j == 0)
    def _():
        o_ref[...] = jnp.zeros_like(o_ref)

    ids = idx_ref[0, 0, :]
    base = j * bn
    cols = jax.lax.broadcasted_iota(jnp.int32, (be, bn), 1) + base
    oh = (ids[:, None] == cols).astype(jnp.float32)
    o_ref[...] += jnp.dot(oh, t_ref[...], preferred_element_type=jnp.float32)


def _gather(idx, table, be=2048, bn=2048):
    ep = idx.shape[0]
    n0, d = table.shape
    np_ = _rup(n0, bn)
    tp = _pad_rows(table, np_)
    idx3 = idx.reshape(ep // be, 1, be)
    return pl.pallas_call(
        functools.partial(_gather_kernel, be=be, bn=bn),
        grid=(ep // be, np_ // bn),
        in_specs=[pl.BlockSpec((1, 1, be), lambda i, j: (i, 0, 0)),
                  pl.BlockSpec((bn, d), lambda i, j: (j, 0))],
        out_specs=pl.BlockSpec((be, d), lambda i, j: (i, 0)),
        out_shape=jax.ShapeDtypeStruct((ep, d), jnp.float32),
    )(idx3, tp)


def _seg_kernel(idx_ref, z_ref, o_ref, *, be, bn):
    j = pl.program_id(1)

    @pl.when(j == 0)
    def _():
        o_ref[...] = jnp.zeros_like(o_ref)

    i = pl.program_id(0)
    ids = idx_ref[0, 0, :]
    base = i * bn
    rows = jax.lax.broadcasted_iota(jnp.int32, (bn, be), 0) + base
    oh = (rows == ids[None, :]).astype(jnp.float32)
    o_ref[...] += jnp.dot(oh, z_ref[...], preferred_element_type=jnp.float32)


def _seg(idx, z, n0, be=2048, bn=2048):
    ep, d = z.shape
    np_ = _rup(n0, bn)
    idx3 = idx.reshape(ep // be, 1, be)
    out = pl.pallas_call(
        functools.partial(_seg_kernel, be=be, bn=bn),
        grid=(np_ // bn, ep // be),
        in_specs=[pl.BlockSpec((1, 1, be), lambda i, j: (j, 0, 0)),
                  pl.BlockSpec((be, d), lambda i, j: (j, 0))],
        out_specs=pl.BlockSpec((bn, d), lambda i, j: (i, 0)),
        out_shape=jax.ShapeDtypeStruct((np_, d), jnp.float32),
    )(idx3, z)
    return out[:n0]


def _head_kernel(x_ref, w1_ref, b1_ref, w2_ref, b2_ref, o_ref, *, nout):
    h = jnp.dot(x_ref[...], w1_ref[...], preferred_element_type=jnp.float32)
    h = _leaky(h + b1_ref[...])
    y = jnp.dot(h, w2_ref[...], preferred_element_type=jnp.float32)
    y = _leaky(y + b2_ref[...])
    mask = jax.lax.broadcasted_iota(jnp.int32, y.shape, 1) < nout
    y = jnp.where(mask, y, -1e30)
    y = y - jnp.max(y, axis=-1, keepdims=True)
    e = jnp.exp(y)
    o_ref[...] = e / jnp.sum(e, axis=-1, keepdims=True)


def _head(x, w1, b1, w2, b2):
    m, k = x.shape
    h = w1.shape[1]
    npad = 128
    w2p = jnp.pad(w2, ((0, 0), (0, npad - w2.shape[1])))
    b2p = jnp.pad(b2, (0, npad - b2.shape[0]))
    out = pl.pallas_call(
        functools.partial(_head_kernel, nout=_OUT),
        grid=(1,),
        in_specs=[pl.BlockSpec((m, k), lambda i: (0, 0)),
                  pl.BlockSpec((k, h), lambda i: (0, 0)),
                  pl.BlockSpec((1, h), lambda i: (0, 0)),
                  pl.BlockSpec((h, npad), lambda i: (0, 0)),
                  pl.BlockSpec((1, npad), lambda i: (0, 0))],
        out_specs=pl.BlockSpec((m, npad), lambda i: (0, 0)),
        out_shape=jax.ShapeDtypeStruct((m, npad), jnp.float32),
    )(x, w1, b1.reshape(1, h), w2p, b2p.reshape(1, npad))
    return out[:, :_OUT]


def kernel(x_tag, x_module, x_question, x_answer, x_comment, ei_0, ei_1, ei_2, ei_3, ei_4, ei_5, ei_6, ei_7, ei_8, ei_9, batch_tag, batch_module, batch_question, batch_answer, batch_comment, post_emb, Wrel_l0_r0, brel_l0_r0, Wroot_l0_r0, Wrel_l0_r1, brel_l0_r1, Wroot_l0_r1, Wrel_l0_r2, brel_l0_r2, Wroot_l0_r2, Wrel_l0_r3, brel_l0_r3, Wroot_l0_r3, Wrel_l0_r4, brel_l0_r4, Wroot_l0_r4, Wrel_l0_r5, brel_l0_r5, Wroot_l0_r5, Wrel_l0_r6, brel_l0_r6, Wroot_l0_r6, Wrel_l0_r7, brel_l0_r7, Wroot_l0_r7, Wrel_l0_r8, brel_l0_r8, Wroot_l0_r8, Wrel_l0_r9, brel_l0_r9, Wroot_l0_r9, Wrel_l1_r0, brel_l1_r0, Wroot_l1_r0, Wrel_l1_r1, brel_l1_r1, Wroot_l1_r1, Wrel_l1_r2, brel_l1_r2, Wroot_l1_r2, Wrel_l1_r3, brel_l1_r3, Wroot_l1_r3, Wrel_l1_r4, brel_l1_r4, Wroot_l1_r4, Wrel_l1_r5, brel_l1_r5, Wroot_l1_r5, Wrel_l1_r6, brel_l1_r6, Wroot_l1_r6, Wrel_l1_r7, brel_l1_r7, Wroot_l1_r7, Wrel_l1_r8, brel_l1_r8, Wroot_l1_r8, Wrel_l1_r9, brel_l1_r9, Wroot_l1_r9, W1, b1, W2, b2):
    d = dict(locals())
    be = 2048
    ep = _rup(_E, be)
    eis = []
    for i in range(len(_RELS)):
        ei = d['ei_%d' % i]
        src = jnp.pad(ei[0], (0, ep - _E))
        dst = jnp.pad(ei[1], (0, ep - _E), constant_values=-1)
        eis.append((src, dst))

    x = {t: d['x_' + t] for t in _TYPES}
    for l in range(_L):
        acc = {t: None for t in _TYPES}
        for i, (s, t) in enumerate(_RELS):
            wrel = d['Wrel_l%d_r%d' % (l, i)]
            brel = d['brel_l%d_r%d' % (l, i)]
            wroot = d['Wroot_l%d_r%d' % (l, i)]
            xw = _mm(x[s], wrel)
            zg = _gather(eis[i][0], xw, be=be)
            agg = _seg(eis[i][1], zg, _NN[t], be=be)
            root = _mm(x[t], wroot, b=brel)
            msg = agg + root
            acc[t] = msg if acc[t] is None else acc[t] + msg
        x = {t: _leaky(acc[t]) for t in _TYPES}

    pooled = []
    for t in _TYPES:
        bidx = d['batch_' + t]
        n = _NN[t]
        npad = _rup(n, be)
        bp = jnp.pad(bidx, (0, npad - n), constant_values=-1)
        zcat = jnp.concatenate(
            [x[t], jnp.ones((n, 1), jnp.float32),
             jnp.zeros((n, 63), jnp.float32)], axis=1)
        zp = _pad_rows(zcat, npad)
        sums = _seg(bp, zp, _B, be=be, bn=1024)
        cnt = sums[:, _H:_H + 1]
        pooled.append(sums[:, :_H] / jnp.clip(cnt, 1.0, None))

    feat = jnp.concatenate(pooled + [d['post_emb']], axis=1)
    return _head(feat, d['W1'], d['b1'], d['W2'], d['b2'])
